# baseline (speedup 1.0000x reference)
import functools

import jax
import jax.numpy as jnp
from jax import lax
from jax.experimental import pallas as pl
from jax.experimental.pallas import tpu as pltpu
from jax.experimental.pallas import tpu_sc as plsc

N = 10000
E = 160000
IN_DIM = 256
H = 8
D = 64
HD = H * D

NC = 2
NS = 16
NW = NC * NS

NSR = 96
SR = 112
NPAD = NSR * SR
EC = 2000
NV = EC // 16
NCHUNK = E // EC
GB = 8
MCAP = 4096
FLUSH_THR = 2080

_MROWS = 10
_MBLK = N // _MROWS


def _proj_body(h_ref, wq_ref, wk_ref, wv_ref, bq_ref, bk_ref, bv_ref,
               q_ref, kv_ref):
    x = h_ref[...]
    q_ref[...] = jnp.dot(x, wq_ref[...], preferred_element_type=jnp.float32) + bq_ref[...]
    kv_ref[:, :HD] = jnp.dot(x, wk_ref[...], preferred_element_type=jnp.float32) + bk_ref[...]
    kv_ref[:, HD:] = jnp.dot(x, wv_ref[...], preferred_element_type=jnp.float32) + bv_ref[...]


def _project(h, wqt, wkt, wvt, bq2, bk2, bv2):
    return pl.pallas_call(
        _proj_body,
        grid=(_MROWS,),
        in_specs=[
            pl.BlockSpec((_MBLK, IN_DIM), lambda i: (i, 0)),
            pl.BlockSpec((IN_DIM, HD), lambda i: (0, 0)),
            pl.BlockSpec((IN_DIM, HD), lambda i: (0, 0)),
            pl.BlockSpec((IN_DIM, HD), lambda i: (0, 0)),
            pl.BlockSpec((1, HD), lambda i: (0, 0)),
            pl.BlockSpec((1, HD), lambda i: (0, 0)),
            pl.BlockSpec((1, HD), lambda i: (0, 0)),
        ],
        out_specs=[
            pl.BlockSpec((_MBLK, HD), lambda i: (i, 0)),
            pl.BlockSpec((_MBLK, 2 * HD), lambda i: (i, 0)),
        ],
        out_shape=[
            jax.ShapeDtypeStruct((N, HD), jnp.float32),
            jax.ShapeDtypeStruct((N, 2 * HD), jnp.float32),
        ],
    )(h, wqt, wkt, wvt, bq2, bk2, bv2)


def _splat(vec, idx):
    return jnp.take_along_axis(
        vec, jnp.full((16,), idx, jnp.int32), axis=0)


def _edge_body(q_hbm, kv_hbm, src_hbm, dst_hbm, zer_hbm, zer2_hbm,
               out_hbm, acc, zacc, dstb, srcb, mdst, msrc,
               kvr0, kvr1, qr0, qr1,
               sem_a0, sem_a1, sem_b0, sem_b1):
    kvr = (kvr0, kvr1)
    qr = (qr0, qr1)
    sem_a = (sem_a0, sem_a1)
    sem_b = (sem_b0, sem_b1)
    wid = lax.axis_index("s") * NC + lax.axis_index("c")
    iota = lax.iota(jnp.int32, 16)
    zeros_i = jnp.zeros((16,), jnp.int32)

    def fire(g, b):
        pltpu.async_copy(
            kv_hbm.at[msrc.at[pl.ds(g * GB, GB)]], kvr[b], sem_a[b])
        pltpu.async_copy(
            q_hbm.at[mdst.at[pl.ds(g * GB, GB)]], qr[b], sem_b[b])

    def drain(g, b):
        pltpu.make_async_copy(
            kv_hbm.at[msrc.at[pl.ds(g * GB, GB)]], kvr[b], sem_a[b]).wait()
        pltpu.make_async_copy(
            q_hbm.at[mdst.at[pl.ds(g * GB, GB)]], qr[b], sem_b[b]).wait()

    def process_group(g, b, lo_v):
        gbase = g * GB
        dl_all = mdst[pl.ds(gbase, 16)] - lo_v

        rotidx = [(iota + sh) & 15 for sh in (1, 2, 4, 8)]

        def quad(qi, _):
            for u in range(4):
                j = qi * 4 + u
                dl = _splat(dl_all, j)
                svec = jnp.zeros((16,), jnp.float32)
                for h in range(H):
                    s4 = jnp.zeros((16,), jnp.float32)
                    for t4 in range(4):
                        t = h * 4 + t4
                        kv = kvr[b][j, pl.ds(16 * t, 16)]
                        qv = qr[b][j, pl.ds(16 * t, 16)]
                        s4 = s4 + kv * qv
                    for ri in rotidx:
                        s4 = s4 + jnp.take_along_axis(s4, ri, axis=0)
                    sp = jnp.exp(jnp.clip(s4 * 0.125, -5.0, 5.0))
                    svec = jnp.where(iota == h, sp, svec)
                    for t4 in range(4):
                        t = h * 4 + t4
                        vv = kvr[b][j, pl.ds(HD + 16 * t, 16)]
                        plsc.addupdate_scatter(
                            acc, [dl, iota + 16 * t], vv * sp)
                plsc.addupdate_scatter(zacc, [dl, iota], svec)
            return 0

        lax.fori_loop(0, GB // 4, quad, 0)

    def process_groups(nf, lo_v):
        @pl.when(nf > 0)
        def _():
            fire(0, 0)

        @pl.when(nf > 1)
        def _():
            fire(1, 1)

        def pair_body(i, _):
            for b in range(2):
                g = 2 * i + b

                @pl.when(g < nf)
                def _(g=g, b=b):
                    drain(g, b)
                    process_group(g, b, lo_v)

                    @pl.when(g + 2 < nf)
                    def _():
                        fire(g + 2, b)

            return 0

        lax.fori_loop(0, (nf + 1) // 2, pair_body, 0)

    def pass_body(p, _):
        sr = wid + NW * p
        lo = sr * SR
        hi = lo + SR
        lo_v = jnp.full((16,), lo, jnp.int32)

        pltpu.sync_copy(zer_hbm, acc.at[pl.ds(0, SR)])
        pltpu.sync_copy(zer2_hbm, zacc.at[pl.ds(0, SR)])

        def chunk_body(c, cnt_vec):
            base_e = c * EC
            pltpu.sync_copy(dst_hbm.at[pl.ds(base_e, EC)], dstb)
            pltpu.sync_copy(src_hbm.at[pl.ds(base_e, EC)], srcb)

            def filt_body(i, cv):
                dv = dstb[pl.ds(i * 16, 16)]
                sv = srcb[pl.ds(i * 16, 16)]
                m = (dv >= lo) & (dv < hi)
                mi = m.astype(jnp.int32)
                pc = plsc.cumsum(mi)
                pos = cv + pc - 1
                plsc.store_scatter(mdst, [pos], dv, mask=m)
                plsc.store_scatter(msrc, [pos], sv, mask=m)
                return cv + plsc.all_reduce_population_count(m)

            cnt_vec = lax.fori_loop(0, NV, filt_body, cnt_vec)
            cnt = jnp.max(cnt_vec)

            def overflow_flush(cnt_vec):
                nf = cnt // GB
                process_groups(nf, lo_v)
                tb = nf * GB
                tail_d = mdst[pl.ds(tb, 16)]
                tail_s = msrc[pl.ds(tb, 16)]
                mdst[pl.ds(0, 16)] = tail_d
                msrc[pl.ds(0, 16)] = tail_s
                return cnt_vec - nf * GB

            return lax.cond(cnt >= FLUSH_THR, overflow_flush,
                            lambda cv: cv, cnt_vec)

        cnt_vec = lax.fori_loop(0, NCHUNK, chunk_body, zeros_i)

        cnt = jnp.max(cnt_vec)
        pos2 = cnt_vec + iota
        plsc.store_scatter(msrc, [pos2], zeros_i)
        plsc.store_scatter(mdst, [pos2], jnp.full((16,), lo + SR, jnp.int32))
        process_groups((cnt + GB - 1) // GB, lo_v)

        def flush_body(i, _):
            zrow = zacc[i, :]
            recv = 1.0 / (zrow + 1e-6)
            for h in range(H):
                rec = _splat(recv, h)
                for t4 in range(4):
                    t = h * 4 + t4
                    a = acc[i, pl.ds(16 * t, 16)]
                    acc[i, pl.ds(16 * t, 16)] = a * rec
            return 0

        lax.fori_loop(0, SR, flush_body, 0)
        pltpu.sync_copy(acc.at[pl.ds(0, SR)], out_hbm.at[pl.ds(lo, SR)])
        return 0

    lax.fori_loop(0, 3, pass_body, 0)


def _edge_phase(q, kv, src, dst):
    zer = jnp.zeros((SR, HD), jnp.float32)
    zer2 = jnp.zeros((SR, 16), jnp.float32)
    mesh = plsc.VectorSubcoreMesh(
        core_axis_name="c", subcore_axis_name="s",
        num_cores=NC, num_subcores=NS)
    fn = pl.kernel(
        _edge_body,
        out_type=jax.ShapeDtypeStruct((NPAD, HD), jnp.float32),
        mesh=mesh,
        compiler_params=pltpu.CompilerParams(needs_layout_passes=False),
        scratch_types=[
            pltpu.VMEM((SR + 1, HD), jnp.float32),
            pltpu.VMEM((SR + 1, 16), jnp.float32),
            pltpu.VMEM((EC,), jnp.int32),
            pltpu.VMEM((EC,), jnp.int32),
            pltpu.VMEM((MCAP,), jnp.int32),
            pltpu.VMEM((MCAP,), jnp.int32),
            pltpu.VMEM((GB, 2 * HD), jnp.float32),
            pltpu.VMEM((GB, 2 * HD), jnp.float32),
            pltpu.VMEM((GB, HD), jnp.float32),
            pltpu.VMEM((GB, HD), jnp.float32),
            pltpu.SemaphoreType.DMA,
            pltpu.SemaphoreType.DMA,
            pltpu.SemaphoreType.DMA,
            pltpu.SemaphoreType.DMA,
        ],
    )
    return fn(q, kv, src, dst, zer, zer2)


def kernel(h, edge_index, Wq, bq, Wk, bk, Wv, bv):
    q, kv = _project(h, Wq.T, Wk.T, Wv.T,
                     bq.reshape(1, HD), bk.reshape(1, HD),
                     bv.reshape(1, HD))
    src = edge_index[0]
    dst = edge_index[1]
    out = _edge_phase(q, kv, src, dst)
    return out[:N].reshape(N, H, D)


# speedup vs baseline: 1.8078x; 1.8078x over previous
import functools

import jax
import jax.numpy as jnp
from jax import lax
from jax.experimental import pallas as pl
from jax.experimental.pallas import tpu as pltpu
from jax.experimental.pallas import tpu_sc as plsc

N = 10000
E = 160000
IN_DIM = 256
H = 8
D = 64
HD = H * D

NC = 2
NS = 16
NW = NC * NS

NSR = 96
SR = 112
NPAD = NSR * SR
EC = 2000
NV = EC // 16
NCHUNK = E // EC
GB = 8
MCAP = 4096
FLUSH_THR = 2080

_MROWS = 10
_MBLK = N // _MROWS


def _proj_body(h_ref, wq_ref, wk_ref, wv_ref, bq_ref, bk_ref, bv_ref,
               q_ref, kv_ref):
    x = h_ref[...]
    q_ref[...] = jnp.dot(x, wq_ref[...], preferred_element_type=jnp.float32) + bq_ref[...]
    kv_ref[:, :HD] = jnp.dot(x, wk_ref[...], preferred_element_type=jnp.float32) + bk_ref[...]
    kv_ref[:, HD:] = jnp.dot(x, wv_ref[...], preferred_element_type=jnp.float32) + bv_ref[...]


def _project(h, wqt, wkt, wvt, bq2, bk2, bv2):
    return pl.pallas_call(
        _proj_body,
        grid=(_MROWS,),
        in_specs=[
            pl.BlockSpec((_MBLK, IN_DIM), lambda i: (i, 0)),
            pl.BlockSpec((IN_DIM, HD), lambda i: (0, 0)),
            pl.BlockSpec((IN_DIM, HD), lambda i: (0, 0)),
            pl.BlockSpec((IN_DIM, HD), lambda i: (0, 0)),
            pl.BlockSpec((1, HD), lambda i: (0, 0)),
            pl.BlockSpec((1, HD), lambda i: (0, 0)),
            pl.BlockSpec((1, HD), lambda i: (0, 0)),
        ],
        out_specs=[
            pl.BlockSpec((_MBLK, HD), lambda i: (i, 0)),
            pl.BlockSpec((_MBLK, 2 * HD), lambda i: (i, 0)),
        ],
        out_shape=[
            jax.ShapeDtypeStruct((N, HD), jnp.float32),
            jax.ShapeDtypeStruct((N, 2 * HD), jnp.float32),
        ],
    )(h, wqt, wkt, wvt, bq2, bk2, bv2)


def _splat(vec, idx):
    return jnp.take_along_axis(
        vec, jnp.full((16,), idx, jnp.int32), axis=0)


def _edge_body(q_hbm, kv_hbm, src_hbm, dst_hbm, zer_hbm, zer2_hbm,
               out_hbm, acc, zacc, dstb, srcb, mdst, msrc,
               kvr0, kvr1, qr0, qr1, stmp,
               sem_a0, sem_a1, sem_b0, sem_b1):
    kvr = (kvr0, kvr1)
    qr = (qr0, qr1)
    sem_a = (sem_a0, sem_a1)
    sem_b = (sem_b0, sem_b1)
    wid = lax.axis_index("s") * NC + lax.axis_index("c")
    iota = lax.iota(jnp.int32, 16)
    zeros_i = jnp.zeros((16,), jnp.int32)

    def fire(g, b):
        pltpu.async_copy(
            kv_hbm.at[msrc.at[pl.ds(g * GB, GB)]], kvr[b], sem_a[b])
        pltpu.async_copy(
            q_hbm.at[mdst.at[pl.ds(g * GB, GB)]], qr[b], sem_b[b])

    def drain(g, b):
        pltpu.make_async_copy(
            kv_hbm.at[msrc.at[pl.ds(g * GB, GB)]], kvr[b], sem_a[b]).wait()
        pltpu.make_async_copy(
            q_hbm.at[mdst.at[pl.ds(g * GB, GB)]], qr[b], sem_b[b]).wait()

    def process_group(g, b, lo_v):
        gbase = g * GB

        def quad(qi, _):
            for u in range(4):
                j = qi * 4 + u
                jv = jnp.full((16,), gbase + j, jnp.int32)
                dl = plsc.load_gather(mdst, [jv]) - lo_v
                svec = jnp.zeros((16,), jnp.float32)
                for h in range(H):
                    s4 = jnp.zeros((16,), jnp.float32)
                    for t4 in range(4):
                        t = h * 4 + t4
                        kv = kvr[b][j, pl.ds(16 * t, 16)]
                        qv = qr[b][j, pl.ds(16 * t, 16)]
                        s4 = s4 + kv * qv
                    sc = jnp.sum(s4)
                    svec = jnp.where(iota == h, sc, svec)
                svec = jnp.exp(jnp.clip(svec * 0.125, -5.0, 5.0))
                plsc.addupdate_scatter(
                    zacc, [dl, iota], jnp.where(iota < H, svec, 0.0))
                soff = 16 * (1 + u)
                stmp[pl.ds(soff, 16)] = svec
                for h in range(H):
                    sp = plsc.load_gather(
                        stmp, [jnp.full((16,), soff + h, jnp.int32)])
                    for t4 in range(4):
                        t = h * 4 + t4
                        vv = kvr[b][j, pl.ds(HD + 16 * t, 16)]
                        plsc.addupdate_scatter(
                            acc, [dl, iota + 16 * t], vv * sp)
            return 0

        lax.fori_loop(0, GB // 4, quad, 0)

    def process_groups(nf, lo_v):
        @pl.when(nf > 0)
        def _():
            fire(0, 0)

        @pl.when(nf > 1)
        def _():
            fire(1, 1)

        def pair_body(i, _):
            for b in range(2):
                g = 2 * i + b

                @pl.when(g < nf)
                def _(g=g, b=b):
                    drain(g, b)
                    process_group(g, b, lo_v)

                    @pl.when(g + 2 < nf)
                    def _():
                        fire(g + 2, b)

            return 0

        lax.fori_loop(0, (nf + 1) // 2, pair_body, 0)

    def pass_body(p, _):
        sr = wid + NW * p
        lo = sr * SR
        hi = lo + SR
        lo_v = jnp.full((16,), lo, jnp.int32)

        pltpu.sync_copy(zer_hbm, acc.at[pl.ds(0, SR)])
        pltpu.sync_copy(zer2_hbm, zacc.at[pl.ds(0, SR)])

        def chunk_body(c, cnt_vec):
            base_e = c * EC
            pltpu.sync_copy(dst_hbm.at[pl.ds(base_e, EC)], dstb)
            pltpu.sync_copy(src_hbm.at[pl.ds(base_e, EC)], srcb)

            def filt_body(i, cv):
                dv = dstb[pl.ds(i * 16, 16)]
                sv = srcb[pl.ds(i * 16, 16)]
                m = (dv >= lo) & (dv < hi)
                mi = m.astype(jnp.int32)
                pc = plsc.cumsum(mi)
                pos = cv + pc - 1
                plsc.store_scatter(mdst, [pos], dv, mask=m)
                plsc.store_scatter(msrc, [pos], sv, mask=m)
                return cv + plsc.all_reduce_population_count(m)

            cnt_vec = lax.fori_loop(0, NV, filt_body, cnt_vec)
            cnt = jnp.max(cnt_vec)

            def overflow_flush(cnt_vec):
                nf = cnt // GB
                process_groups(nf, lo_v)
                tb = nf * GB
                tail_d = mdst[pl.ds(tb, 16)]
                tail_s = msrc[pl.ds(tb, 16)]
                mdst[pl.ds(0, 16)] = tail_d
                msrc[pl.ds(0, 16)] = tail_s
                return cnt_vec - nf * GB

            return lax.cond(cnt >= FLUSH_THR, overflow_flush,
                            lambda cv: cv, cnt_vec)

        cnt_vec = lax.fori_loop(0, NCHUNK, chunk_body, zeros_i)

        cnt = jnp.max(cnt_vec)
        pos2 = cnt_vec + iota
        plsc.store_scatter(msrc, [pos2], zeros_i)
        plsc.store_scatter(mdst, [pos2], jnp.full((16,), lo + SR, jnp.int32))
        process_groups((cnt + GB - 1) // GB, lo_v)

        def flush_body(i, _):
            zrow = zacc[i, :]
            recv = 1.0 / (zrow + 1e-6)
            for h in range(H):
                rec = _splat(recv, h)
                for t4 in range(4):
                    t = h * 4 + t4
                    a = acc[i, pl.ds(16 * t, 16)]
                    acc[i, pl.ds(16 * t, 16)] = a * rec
            return 0

        lax.fori_loop(0, SR, flush_body, 0)
        pltpu.sync_copy(acc.at[pl.ds(0, SR)], out_hbm.at[pl.ds(lo, SR)])
        return 0

    lax.fori_loop(0, 3, pass_body, 0)


def _edge_phase(q, kv, src, dst):
    zer = jnp.zeros((SR, HD), jnp.float32)
    zer2 = jnp.zeros((SR, 16), jnp.float32)
    mesh = plsc.VectorSubcoreMesh(
        core_axis_name="c", subcore_axis_name="s",
        num_cores=NC, num_subcores=NS)
    fn = pl.kernel(
        _edge_body,
        out_type=jax.ShapeDtypeStruct((NPAD, HD), jnp.float32),
        mesh=mesh,
        compiler_params=pltpu.CompilerParams(needs_layout_passes=False),
        scratch_types=[
            pltpu.VMEM((SR + 1, HD), jnp.float32),
            pltpu.VMEM((SR + 1, 16), jnp.float32),
            pltpu.VMEM((EC,), jnp.int32),
            pltpu.VMEM((EC,), jnp.int32),
            pltpu.VMEM((MCAP,), jnp.int32),
            pltpu.VMEM((MCAP,), jnp.int32),
            pltpu.VMEM((GB, 2 * HD), jnp.float32),
            pltpu.VMEM((GB, 2 * HD), jnp.float32),
            pltpu.VMEM((GB, HD), jnp.float32),
            pltpu.VMEM((GB, HD), jnp.float32),
            pltpu.VMEM((128,), jnp.float32),
            pltpu.SemaphoreType.DMA,
            pltpu.SemaphoreType.DMA,
            pltpu.SemaphoreType.DMA,
            pltpu.SemaphoreType.DMA,
        ],
    )
    return fn(q, kv, src, dst, zer, zer2)


def kernel(h, edge_index, Wq, bq, Wk, bk, Wv, bv):
    q, kv = _project(h, Wq.T, Wk.T, Wv.T,
                     bq.reshape(1, HD), bk.reshape(1, HD),
                     bv.reshape(1, HD))
    src = edge_index[0]
    dst = edge_index[1]
    out = _edge_phase(q, kv, src, dst)
    return out[:N].reshape(N, H, D)


# speedup vs baseline: 1.9638x; 1.0863x over previous
import functools

import jax
import jax.numpy as jnp
from jax import lax
from jax.experimental import pallas as pl
from jax.experimental.pallas import tpu as pltpu
from jax.experimental.pallas import tpu_sc as plsc

N = 10000
E = 160000
IN_DIM = 256
H = 8
D = 64
HD = H * D

NC = 2
NS = 16
NW = NC * NS

NSR = 96
SR = 112
NPAD = NSR * SR
EC = 4000
NV = EC // 16
NCHUNK = E // EC
GB = 8
MCAP = 8192
FLUSH_THR = 2080

_MROWS = 10
_MBLK = N // _MROWS


def _proj_body(h_ref, wq_ref, wk_ref, wv_ref, bq_ref, bk_ref, bv_ref,
               q_ref, kv_ref):
    x = h_ref[...]
    q_ref[...] = jnp.dot(x, wq_ref[...], preferred_element_type=jnp.float32) + bq_ref[...]
    kv_ref[:, :HD] = jnp.dot(x, wk_ref[...], preferred_element_type=jnp.float32) + bk_ref[...]
    kv_ref[:, HD:] = jnp.dot(x, wv_ref[...], preferred_element_type=jnp.float32) + bv_ref[...]


def _project(h, wqt, wkt, wvt, bq2, bk2, bv2):
    return pl.pallas_call(
        _proj_body,
        grid=(_MROWS,),
        in_specs=[
            pl.BlockSpec((_MBLK, IN_DIM), lambda i: (i, 0)),
            pl.BlockSpec((IN_DIM, HD), lambda i: (0, 0)),
            pl.BlockSpec((IN_DIM, HD), lambda i: (0, 0)),
            pl.BlockSpec((IN_DIM, HD), lambda i: (0, 0)),
            pl.BlockSpec((1, HD), lambda i: (0, 0)),
            pl.BlockSpec((1, HD), lambda i: (0, 0)),
            pl.BlockSpec((1, HD), lambda i: (0, 0)),
        ],
        out_specs=[
            pl.BlockSpec((_MBLK, HD), lambda i: (i, 0)),
            pl.BlockSpec((_MBLK, 2 * HD), lambda i: (i, 0)),
        ],
        out_shape=[
            jax.ShapeDtypeStruct((N, HD), jnp.float32),
            jax.ShapeDtypeStruct((N, 2 * HD), jnp.float32),
        ],
    )(h, wqt, wkt, wvt, bq2, bk2, bv2)


def _splat(vec, idx):
    return jnp.take_along_axis(
        vec, jnp.full((16,), idx, jnp.int32), axis=0)


def _edge_body(q_hbm, kv_hbm, src_hbm, dst_hbm, zer_hbm, zer2_hbm,
               out_hbm, acc, zacc, dstb, srcb, mdst, msrc,
               kvr0, kvr1, qr0, qr1,
               sem_a0, sem_a1, sem_b0, sem_b1):
    kvr = (kvr0, kvr1)
    qr = (qr0, qr1)
    sem_a = (sem_a0, sem_a1)
    sem_b = (sem_b0, sem_b1)
    wid = lax.axis_index("s") * NC + lax.axis_index("c")
    iota = lax.iota(jnp.int32, 16)
    zeros_i = jnp.zeros((16,), jnp.int32)

    def fire(g, b):
        pltpu.async_copy(
            kv_hbm.at[msrc.at[pl.ds(g * GB, GB)]], kvr[b], sem_a[b])
        pltpu.async_copy(
            q_hbm.at[mdst.at[pl.ds(g * GB, GB)]], qr[b], sem_b[b])

    def drain(g, b):
        pltpu.make_async_copy(
            kv_hbm.at[msrc.at[pl.ds(g * GB, GB)]], kvr[b], sem_a[b]).wait()
        pltpu.make_async_copy(
            q_hbm.at[mdst.at[pl.ds(g * GB, GB)]], qr[b], sem_b[b]).wait()

    def process_group(g, b, lo_v):
        gbase = g * GB
        dl_all = mdst[pl.ds(gbase, 16)] - lo_v

        def quad(qi, _):
            for u in range(4):
                j = qi * 4 + u
                dl = _splat(dl_all, j)
                svec = jnp.zeros((16,), jnp.float32)
                for h in range(H):
                    s4 = jnp.zeros((16,), jnp.float32)
                    for t4 in range(4):
                        t = h * 4 + t4
                        kv = kvr[b][j, pl.ds(16 * t, 16)]
                        qv = qr[b][j, pl.ds(16 * t, 16)]
                        s4 = s4 + kv * qv
                    sc = jnp.sum(s4)
                    svec = jnp.where(iota == h, sc, svec)
                svec = jnp.exp(jnp.clip(svec * 0.125, -5.0, 5.0))
                plsc.addupdate_scatter(
                    zacc, [dl, iota], jnp.where(iota < H, svec, 0.0))
                for h in range(H):
                    sp = _splat(svec, h)
                    for t4 in range(4):
                        t = h * 4 + t4
                        vv = kvr[b][j, pl.ds(HD + 16 * t, 16)]
                        plsc.addupdate_scatter(
                            acc, [dl, iota + 16 * t], vv * sp)
            return 0

        lax.fori_loop(0, GB // 4, quad, 0)

    def process_groups(nf, lo_v):
        @pl.when(nf > 0)
        def _():
            fire(0, 0)

        @pl.when(nf > 1)
        def _():
            fire(1, 1)

        def pair_body(i, _):
            for b in range(2):
                g = 2 * i + b

                @pl.when(g < nf)
                def _(g=g, b=b):
                    drain(g, b)
                    process_group(g, b, lo_v)

                    @pl.when(g + 2 < nf)
                    def _():
                        fire(g + 2, b)

            return 0

        lax.fori_loop(0, (nf + 1) // 2, pair_body, 0)

    def pass_body(p, _):
        sr = wid + NW * p
        lo = sr * SR
        hi = lo + SR
        lo_v = jnp.full((16,), lo, jnp.int32)

        pltpu.sync_copy(zer_hbm, acc.at[pl.ds(0, SR)])
        pltpu.sync_copy(zer2_hbm, zacc.at[pl.ds(0, SR)])

        def chunk_body(c, cnt_vec):
            base_e = c * EC
            pltpu.sync_copy(dst_hbm.at[pl.ds(base_e, EC)], dstb)
            pltpu.sync_copy(src_hbm.at[pl.ds(base_e, EC)], srcb)

            def filt_body(i, cv):
                dv = dstb[pl.ds(i * 16, 16)]
                sv = srcb[pl.ds(i * 16, 16)]
                m = (dv >= lo) & (dv < hi)
                mi = m.astype(jnp.int32)
                pc = plsc.cumsum(mi)
                pos = cv + pc - 1
                plsc.store_scatter(mdst, [pos], dv, mask=m)
                plsc.store_scatter(msrc, [pos], sv, mask=m)
                return cv + plsc.all_reduce_population_count(m)

            cnt_vec = lax.fori_loop(0, NV, filt_body, cnt_vec)
            cnt = jnp.max(cnt_vec)

            def overflow_flush(cnt_vec):
                nf = cnt // GB
                process_groups(nf, lo_v)
                tb = nf * GB
                tail_d = mdst[pl.ds(tb, 16)]
                tail_s = msrc[pl.ds(tb, 16)]
                mdst[pl.ds(0, 16)] = tail_d
                msrc[pl.ds(0, 16)] = tail_s
                return cnt_vec - nf * GB

            return lax.cond(cnt >= FLUSH_THR, overflow_flush,
                            lambda cv: cv, cnt_vec)

        cnt_vec = lax.fori_loop(0, NCHUNK, chunk_body, zeros_i)

        cnt = jnp.max(cnt_vec)
        pos2 = cnt_vec + iota
        plsc.store_scatter(msrc, [pos2], zeros_i)
        plsc.store_scatter(mdst, [pos2], jnp.full((16,), lo + SR, jnp.int32))
        process_groups((cnt + GB - 1) // GB, lo_v)

        def flush_body(i, _):
            zrow = zacc[i, :]
            recv = 1.0 / (zrow + 1e-6)
            for h in range(H):
                rec = _splat(recv, h)
                for t4 in range(4):
                    t = h * 4 + t4
                    a = acc[i, pl.ds(16 * t, 16)]
                    acc[i, pl.ds(16 * t, 16)] = a * rec
            return 0

        lax.fori_loop(0, SR, flush_body, 0)
        pltpu.sync_copy(acc.at[pl.ds(0, SR)], out_hbm.at[pl.ds(lo, SR)])
        return 0

    lax.fori_loop(0, 3, pass_body, 0)


def _edge_phase(q, kv, src, dst):
    zer = jnp.zeros((SR, HD), jnp.float32)
    zer2 = jnp.zeros((SR, 16), jnp.float32)
    mesh = plsc.VectorSubcoreMesh(
        core_axis_name="c", subcore_axis_name="s",
        num_cores=NC, num_subcores=NS)
    fn = pl.kernel(
        _edge_body,
        out_type=jax.ShapeDtypeStruct((NPAD, HD), jnp.float32),
        mesh=mesh,
        compiler_params=pltpu.CompilerParams(needs_layout_passes=False),
        scratch_types=[
            pltpu.VMEM((SR + 1, HD), jnp.float32),
            pltpu.VMEM((SR + 1, 16), jnp.float32),
            pltpu.VMEM((EC,), jnp.int32),
            pltpu.VMEM((EC,), jnp.int32),
            pltpu.VMEM((MCAP,), jnp.int32),
            pltpu.VMEM((MCAP,), jnp.int32),
            pltpu.VMEM((GB, 2 * HD), jnp.float32),
            pltpu.VMEM((GB, 2 * HD), jnp.float32),
            pltpu.VMEM((GB, HD), jnp.float32),
            pltpu.VMEM((GB, HD), jnp.float32),
            pltpu.SemaphoreType.DMA,
            pltpu.SemaphoreType.DMA,
            pltpu.SemaphoreType.DMA,
            pltpu.SemaphoreType.DMA,
        ],
    )
    return fn(q, kv, src, dst, zer, zer2)


def kernel(h, edge_index, Wq, bq, Wk, bk, Wv, bv):
    q, kv = _project(h, Wq.T, Wk.T, Wv.T,
                     bq.reshape(1, HD), bk.reshape(1, HD),
                     bv.reshape(1, HD))
    src = edge_index[0]
    dst = edge_index[1]
    out = _edge_phase(q, kv, src, dst)
    return out[:N].reshape(N, H, D)
